# baseline (speedup 1.0000x reference)
import functools

import jax
import jax.numpy as jnp
from jax import lax
from jax.experimental import pallas as pl
from jax.experimental.pallas import tpu as pltpu
from jax.experimental.pallas import tpu_sc as plsc

_NC = 2
_NS = 16
_L = 16


def _make_sc_kernel(S, B, D, CS):
    n_workers = _NC * _NS
    rows_per_w = S // n_workers
    n_chunks = rows_per_w // CS
    mesh = plsc.VectorSubcoreMesh(
        core_axis_name="c", subcore_axis_name="s",
        num_cores=_NC, num_subcores=_NS,
    )

    nbuf = 6

    @functools.partial(
        pl.kernel,
        out_type=jax.ShapeDtypeStruct((S, B, D), jnp.float32),
        mesh=mesh,
        scratch_types=(
            [pltpu.VMEM((CS, B, D), jnp.float32) for _ in range(nbuf)]
            + [pltpu.VMEM((CS, D), jnp.float32) for _ in range(nbuf)]
            + [pltpu.SemaphoreType.DMA] * (3 * nbuf)
        ),
    )
    def sc_kernel(x_hbm, pos_hbm, out_hbm, *refs):
        xb = refs[0:nbuf]
        pb = refs[nbuf:2 * nbuf]
        slx = refs[2 * nbuf:3 * nbuf]
        slp = refs[3 * nbuf:4 * nbuf]
        sst = refs[4 * nbuf:5 * nbuf]

        wid = lax.axis_index("s") * _NC + lax.axis_index("c")
        base = wid * rows_per_w

        def start_load(j, b):
            s0 = base + j * CS
            pltpu.async_copy(x_hbm.at[pl.ds(s0, CS)], xb[b], slx[b])
            pltpu.async_copy(pos_hbm.at[pl.ds(s0, CS)], pb[b], slp[b])

        def wait_load(b):
            pltpu.make_async_copy(x_hbm.at[pl.ds(0, CS)], xb[b], slx[b]).wait()
            pltpu.make_async_copy(pos_hbm.at[pl.ds(0, CS)], pb[b], slp[b]).wait()

        def start_store(j, b):
            s0 = base + j * CS
            pltpu.async_copy(xb[b], out_hbm.at[pl.ds(s0, CS)], sst[b])

        def wait_store(b):
            pltpu.make_async_copy(xb[b], out_hbm.at[pl.ds(0, CS)], sst[b]).wait()

        def compute(b):
            @pl.loop(0, D // _L)
            def _vec(k):
                sl = pl.ds(k * _L, _L)
                for r in range(CS):
                    p = pb[b][r, sl]
                    for bb in range(B):
                        xb[b][r, bb, sl] = xb[b][r, bb, sl] + p

        depth = 3

        def step(j, b):
            lb = (b + depth) % nbuf

            def issue():
                if isinstance(j, int):
                    if j + depth >= nbuf:
                        wait_store(lb)
                    start_load(j + depth, lb)
                else:
                    @pl.when(j + depth >= nbuf)
                    def _ws():
                        wait_store(lb)
                    start_load(j + depth, lb)

            if isinstance(j, int):
                if j + depth < n_chunks:
                    issue()
            else:
                @pl.when(j + depth < n_chunks)
                def _ld():
                    issue()

            wait_load(b)
            compute(b)
            start_store(j, b)

        for j0 in range(depth):
            start_load(j0, j0)
        n_loop = (n_chunks // nbuf) * nbuf

        @pl.loop(0, n_loop, step=nbuf)
        def _chunk(i):
            for b in range(nbuf):
                step(i + b, b)

        for jt in range(n_loop, n_chunks):
            step(jt, jt % nbuf)

        for jt in range(n_chunks - nbuf, n_chunks):
            wait_store(jt % nbuf)

    return sc_kernel


def kernel(x, pos_table):
    S, B, D = x.shape
    return _make_sc_kernel(S, B, D, CS=2)(x, pos_table)


# speedup vs baseline: 1.0864x; 1.0864x over previous
import functools

import jax
import jax.numpy as jnp
from jax import lax
from jax.experimental import pallas as pl
from jax.experimental.pallas import tpu as pltpu
from jax.experimental.pallas import tpu_sc as plsc

_NC = 2
_NS = 16
_L = 16


def _make_sc_kernel(S, B, D, CS):
    n_workers = _NC * _NS
    rows_per_w = S // n_workers
    n_chunks = rows_per_w // CS
    mesh = plsc.VectorSubcoreMesh(
        core_axis_name="c", subcore_axis_name="s",
        num_cores=_NC, num_subcores=_NS,
    )

    nbuf = 3

    @functools.partial(
        pl.kernel,
        out_type=jax.ShapeDtypeStruct((S, B, D), jnp.float32),
        mesh=mesh,
        scratch_types=(
            [pltpu.VMEM((CS, B, D), jnp.float32) for _ in range(nbuf)]
            + [pltpu.VMEM((CS, D), jnp.float32) for _ in range(nbuf)]
            + [pltpu.SemaphoreType.DMA] * (3 * nbuf)
        ),
    )
    def sc_kernel(x_hbm, pos_hbm, out_hbm, *refs):
        xb = refs[0:nbuf]
        pb = refs[nbuf:2 * nbuf]
        slx = refs[2 * nbuf:3 * nbuf]
        slp = refs[3 * nbuf:4 * nbuf]
        sst = refs[4 * nbuf:5 * nbuf]

        wid = lax.axis_index("s") * _NC + lax.axis_index("c")
        base = wid * rows_per_w

        def start_load(j, b):
            s0 = base + j * CS
            pltpu.async_copy(x_hbm.at[pl.ds(s0, CS)], xb[b], slx[b])
            pltpu.async_copy(pos_hbm.at[pl.ds(s0, CS)], pb[b], slp[b])

        def wait_load(b):
            pltpu.make_async_copy(x_hbm.at[pl.ds(0, CS)], xb[b], slx[b]).wait()
            pltpu.make_async_copy(pos_hbm.at[pl.ds(0, CS)], pb[b], slp[b]).wait()

        def start_store(j, b):
            s0 = base + j * CS
            pltpu.async_copy(xb[b], out_hbm.at[pl.ds(s0, CS)], sst[b])

        def wait_store(b):
            pltpu.make_async_copy(xb[b], out_hbm.at[pl.ds(0, CS)], sst[b]).wait()

        def compute(b):
            @pl.loop(0, D // _L)
            def _vec(k):
                sl = pl.ds(k * _L, _L)
                for r in range(CS):
                    p = pb[b][r, sl]
                    for bb in range(B):
                        xb[b][r, bb, sl] = xb[b][r, bb, sl] + p

        depth = 2

        def step(j, b):
            lb = (b + depth) % nbuf

            def issue():
                if isinstance(j, int):
                    if j + depth >= nbuf:
                        wait_store(lb)
                    start_load(j + depth, lb)
                else:
                    @pl.when(j + depth >= nbuf)
                    def _ws():
                        wait_store(lb)
                    start_load(j + depth, lb)

            if isinstance(j, int):
                if j + depth < n_chunks:
                    issue()
            else:
                @pl.when(j + depth < n_chunks)
                def _ld():
                    issue()

            wait_load(b)
            compute(b)
            start_store(j, b)

        for j0 in range(depth):
            start_load(j0, j0)
        n_loop = (n_chunks // nbuf) * nbuf

        @pl.loop(0, n_loop, step=nbuf)
        def _chunk(i):
            for b in range(nbuf):
                step(i + b, b)

        for jt in range(n_loop, n_chunks):
            step(jt, jt % nbuf)

        for jt in range(n_chunks - nbuf, n_chunks):
            wait_store(jt % nbuf)

    return sc_kernel


def kernel(x, pos_table):
    S, B, D = x.shape
    return _make_sc_kernel(S, B, D, CS=4)(x, pos_table)


# speedup vs baseline: 1.0864x; 1.0000x over previous
import functools

import jax
import jax.numpy as jnp
from jax import lax
from jax.experimental import pallas as pl
from jax.experimental.pallas import tpu as pltpu
from jax.experimental.pallas import tpu_sc as plsc

_NC = 2
_NS = 16
_L = 16


def _make_sc_kernel(S, B, D, CS):
    n_workers = _NC * _NS
    rows_per_w = S // n_workers
    n_chunks = rows_per_w // CS
    mesh = plsc.VectorSubcoreMesh(
        core_axis_name="c", subcore_axis_name="s",
        num_cores=_NC, num_subcores=_NS,
    )

    nbuf = 3

    @functools.partial(
        pl.kernel,
        out_type=jax.ShapeDtypeStruct((S, B, D), jnp.float32),
        mesh=mesh,
        scratch_types=(
            [pltpu.VMEM((CS, B, D), jnp.float32) for _ in range(nbuf)]
            + [pltpu.VMEM((CS, D), jnp.float32) for _ in range(nbuf)]
            + [pltpu.SemaphoreType.DMA] * (3 * nbuf)
        ),
    )
    def sc_kernel(x_hbm, pos_hbm, out_hbm, *refs):
        xb = refs[0:nbuf]
        pb = refs[nbuf:2 * nbuf]
        slx = refs[2 * nbuf:3 * nbuf]
        slp = refs[3 * nbuf:4 * nbuf]
        sst = refs[4 * nbuf:5 * nbuf]

        wid = lax.axis_index("s") * _NC + lax.axis_index("c")
        base = wid * rows_per_w

        h = CS // 2

        def start_load(j, b):
            s0 = base + j * CS
            pltpu.async_copy(x_hbm.at[pl.ds(s0, h)], xb[b].at[pl.ds(0, h)],
                             slx[b])
            pltpu.async_copy(x_hbm.at[pl.ds(s0 + h, h)], xb[b].at[pl.ds(h, h)],
                             slx[b])
            pltpu.async_copy(pos_hbm.at[pl.ds(s0, CS)], pb[b], slp[b])

        def wait_load(b):
            pltpu.make_async_copy(x_hbm.at[pl.ds(0, CS)], xb[b], slx[b]).wait()
            pltpu.make_async_copy(pos_hbm.at[pl.ds(0, CS)], pb[b], slp[b]).wait()

        def start_store(j, b):
            s0 = base + j * CS
            pltpu.async_copy(xb[b], out_hbm.at[pl.ds(s0, CS)], sst[b])

        def wait_store(b):
            pltpu.make_async_copy(xb[b], out_hbm.at[pl.ds(0, CS)], sst[b]).wait()

        def compute(b):
            @pl.loop(0, D // _L)
            def _vec(k):
                sl = pl.ds(k * _L, _L)
                for r in range(CS):
                    p = pb[b][r, sl]
                    for bb in range(B):
                        xb[b][r, bb, sl] = xb[b][r, bb, sl] + p

        depth = 2

        def step(j, b):
            lb = (b + depth) % nbuf

            def issue():
                if isinstance(j, int):
                    if j + depth >= nbuf:
                        wait_store(lb)
                    start_load(j + depth, lb)
                else:
                    @pl.when(j + depth >= nbuf)
                    def _ws():
                        wait_store(lb)
                    start_load(j + depth, lb)

            if isinstance(j, int):
                if j + depth < n_chunks:
                    issue()
            else:
                @pl.when(j + depth < n_chunks)
                def _ld():
                    issue()

            wait_load(b)
            compute(b)
            start_store(j, b)

        for j0 in range(depth):
            start_load(j0, j0)
        n_loop = (n_chunks // nbuf) * nbuf

        @pl.loop(0, n_loop, step=nbuf)
        def _chunk(i):
            for b in range(nbuf):
                step(i + b, b)

        for jt in range(n_loop, n_chunks):
            step(jt, jt % nbuf)

        for jt in range(n_chunks - nbuf, n_chunks):
            wait_store(jt % nbuf)

    return sc_kernel


def kernel(x, pos_table):
    S, B, D = x.shape
    return _make_sc_kernel(S, B, D, CS=4)(x, pos_table)


# speedup vs baseline: 1.0923x; 1.0054x over previous
import functools

import jax
import jax.numpy as jnp
from jax import lax
from jax.experimental import pallas as pl
from jax.experimental.pallas import tpu as pltpu
from jax.experimental.pallas import tpu_sc as plsc

_NC = 2
_NS = 16
_L = 16


def _make_sc_kernel(S, B, D, CS):
    n_workers = _NC * _NS
    rows_per_w = S // n_workers
    n_chunks = rows_per_w // CS
    mesh = plsc.VectorSubcoreMesh(
        core_axis_name="c", subcore_axis_name="s",
        num_cores=_NC, num_subcores=_NS,
    )

    nbuf = 3

    @functools.partial(
        pl.kernel,
        out_type=jax.ShapeDtypeStruct((S, B, D), jnp.float32),
        mesh=mesh,
        scratch_types=(
            [pltpu.VMEM((CS, B, D), jnp.float32) for _ in range(nbuf)]
            + [pltpu.VMEM((CS, D), jnp.float32) for _ in range(nbuf)]
            + [pltpu.SemaphoreType.DMA] * (3 * nbuf)
        ),
    )
    def sc_kernel(x_hbm, pos_hbm, out_hbm, *refs):
        xb = refs[0:nbuf]
        pb = refs[nbuf:2 * nbuf]
        slx = refs[2 * nbuf:3 * nbuf]
        slp = refs[3 * nbuf:4 * nbuf]
        sst = refs[4 * nbuf:5 * nbuf]

        wid = lax.axis_index("s") * _NC + lax.axis_index("c")
        base = wid * rows_per_w

        def start_load(j, b):
            s0 = base + j * CS
            pltpu.async_copy(x_hbm.at[pl.ds(s0, CS)], xb[b], slx[b])
            pltpu.async_copy(pos_hbm.at[pl.ds(s0, CS)], pb[b], slp[b])

        def wait_load(b):
            pltpu.make_async_copy(x_hbm.at[pl.ds(0, CS)], xb[b], slx[b]).wait()
            pltpu.make_async_copy(pos_hbm.at[pl.ds(0, CS)], pb[b], slp[b]).wait()

        def start_store(j, b):
            s0 = base + j * CS
            pltpu.async_copy(xb[b], out_hbm.at[pl.ds(s0, CS)], sst[b])

        def wait_store(b):
            pltpu.make_async_copy(xb[b], out_hbm.at[pl.ds(0, CS)], sst[b]).wait()

        def compute(b):
            @pl.loop(0, D // _L)
            def _vec(k):
                sl = pl.ds(k * _L, _L)
                for r in range(CS):
                    p = pb[b][r, sl]
                    for bb in range(B):
                        xb[b][r, bb, sl] = xb[b][r, bb, sl] + p

        depth = 2

        def step(j, b):
            lb = (b + depth) % nbuf

            def issue():
                if isinstance(j, int):
                    if j + depth >= nbuf:
                        wait_store(lb)
                    start_load(j + depth, lb)
                else:
                    @pl.when(j + depth >= nbuf)
                    def _ws():
                        wait_store(lb)
                    start_load(j + depth, lb)

            if isinstance(j, int):
                if j + depth < n_chunks:
                    issue()
            else:
                @pl.when(j + depth < n_chunks)
                def _ld():
                    issue()

            wait_load(b)
            compute(b)
            start_store(j, b)

        for j0 in range(depth):
            start_load(j0, j0)
        n_loop = (n_chunks // nbuf) * nbuf

        @pl.loop(0, n_loop, step=nbuf)
        def _chunk(i):
            for b in range(nbuf):
                step(i + b, b)

        for jt in range(n_loop, n_chunks):
            step(jt, jt % nbuf)

        for jt in range(n_chunks - nbuf, n_chunks):
            wait_store(jt % nbuf)

    return sc_kernel


def kernel(x, pos_table):
    S, B, D = x.shape
    return _make_sc_kernel(S, B, D, CS=4)(x, pos_table)
